# baseline (speedup 1.0000x reference)
import functools

import jax
import jax.numpy as jnp
from jax import lax
from jax.experimental import pallas as pl
from jax.experimental.pallas import tpu as pltpu
from jax.experimental.pallas import tpu_sc as plsc

NN = 10000
NP = 10240
EE = 320000
NSUB = 16
CHUNK = 128
NCD = 81
NC = 81
EPAD = 2 * NSUB * NC * CHUNK
BN_EPS = 1e-3

_mesh = plsc.VectorSubcoreMesh(core_axis_name="c", subcore_axis_name="s")

_BCAST_DNUMS = lax.GatherDimensionNumbers(
    offset_dims=(), collapsed_slice_dims=(0,), start_index_map=(0,))


def _bcast_lane(v16, lane):
    idx = jnp.full((16, 1), lane, jnp.int32)
    return lax.gather(v16, idx, _BCAST_DNUMS, (1,),
                      mode=lax.GatherScatterMode.PROMISE_IN_BOUNDS)


@functools.partial(
    pl.kernel,
    out_type=jax.ShapeDtypeStruct((2, NSUB, 640), jnp.float32),
    mesh=_mesh,
    scratch_types=[
        pltpu.VMEM((NCD, 128), jnp.int32),
        pltpu.VMEM((NCD, 128), jnp.float32),
        pltpu.VMEM((640,), jnp.float32),
        pltpu.VMEM_SHARED((NP,), jnp.float32),
    ],
)
def _deg_kernel(col_hbm, ew_hbm, out_hbm, col_v, ew_v, zb, acc):
    cid = lax.axis_index("c")
    sid = lax.axis_index("s")
    zeros16 = jnp.zeros((16,), jnp.float32)

    def zb_body(i, _):
        zb[pl.ds(i * 16, 16)] = zeros16
        return 0

    lax.fori_loop(0, 640 // 16, zb_body, 0)
    pltpu.sync_copy(zb, acc.at[pl.ds(sid * 640, 640)])
    pltpu.sync_copy(col_hbm.at[cid, sid], col_v)
    pltpu.sync_copy(ew_hbm.at[cid, sid], ew_v)
    plsc.subcore_barrier()

    def chunk_body(j, _):
        pltpu.sync_copy(ew_v.at[j], acc.at[col_v.at[j]], add=True)
        return 0

    lax.fori_loop(0, NCD, chunk_body, 0)
    plsc.subcore_barrier()
    pltpu.sync_copy(acc.at[pl.ds(sid * 640, 640)], out_hbm.at[cid, sid])


@functools.partial(
    pl.kernel,
    out_type=jax.ShapeDtypeStruct((2, NSUB, 640, 128), jnp.float32),
    mesh=_mesh,
    scratch_types=[
        pltpu.VMEM((6, CHUNK), jnp.int32),
        pltpu.VMEM((6, CHUNK), jnp.int32),
        pltpu.VMEM((6, CHUNK), jnp.float32),
        pltpu.VMEM((CHUNK, 128), jnp.float32),
        pltpu.VMEM((CHUNK, 128), jnp.float32),
        pltpu.VMEM_SHARED((NP, 128), jnp.float32),
        pltpu.SemaphoreType.DMA,
        pltpu.SemaphoreType.DMA,
        pltpu.SemaphoreType.DMA,
        pltpu.SemaphoreType.DMA,
        pltpu.SemaphoreType.DMA,
        pltpu.SemaphoreType.DMA,
        pltpu.SemaphoreType.DMA,
        pltpu.SemaphoreType.DMA,
        pltpu.SemaphoreType.DMA,
        pltpu.SemaphoreType.DMA,
        pltpu.SemaphoreType.DMA,
        pltpu.SemaphoreType.DMA,
    ],
)
def _agg_kernel(us_hbm, epr_hbm, epc_hbm, epw_hbm, out_hbm,
                err_, erc, erw, b0, b1, acc,
                es0, es1, es2, es3, es4, es5,
                gs0, gs1, gs2, ss0, ss1, ss2):
    cid = lax.axis_index("c")
    sid = lax.axis_index("s")
    bufs = (b0, b1)
    esems = (es0, es1, es2, es3, es4, es5)
    gsems = (gs0, gs1, gs2)
    ssems = (ss0, ss1, ss2)
    wid = cid * NSUB + sid
    zeros16 = jnp.zeros((16,), jnp.float32)

    def e_start(j, r):
        pltpu.async_copy(epr_hbm.at[wid, j], err_.at[r], esems[r])
        pltpu.async_copy(epc_hbm.at[wid, j], erc.at[r], esems[r])
        pltpu.async_copy(epw_hbm.at[wid, j], erw.at[r], esems[r])

    def e_wait(r):
        pltpu.make_async_copy(epr_hbm.at[wid, 0], err_.at[r], esems[r]).wait()
        pltpu.make_async_copy(epc_hbm.at[wid, 0], erc.at[r], esems[r]).wait()
        pltpu.make_async_copy(epw_hbm.at[wid, 0], erw.at[r], esems[r]).wait()

    def g_start(r, b):
        pltpu.async_copy(us_hbm.at[err_.at[r]], bufs[b], gsems[b])

    def g_wait(b):
        pltpu.make_async_copy(us_hbm.at[err_.at[0]], bufs[b], gsems[b]).wait()

    def s_start(r, b):
        pltpu.async_copy(bufs[b], acc.at[erc.at[r]], ssems[b], add=True)

    def s_wait(b):
        pltpu.make_async_copy(bufs[b], acc.at[erc.at[0]], ssems[b]).wait()

    def scale(r, b):
        buf = bufs[b]

        def escale(g, _):
            ew16 = erw[r, pl.ds(g * 16, 16)]
            for e16 in range(16):
                w = _bcast_lane(ew16, e16)
                e = g * 16 + e16
                for f in range(128 // 16):
                    sl = pl.ds(f * 16, 16)
                    buf[e, sl] = buf[e, sl] * w
            return 0

        lax.fori_loop(0, CHUNK // 16, escale, 0)

    def zrow(i, _):
        for f in range(128 // 16):
            b0[i, pl.ds(f * 16, 16)] = zeros16
        return 0

    lax.fori_loop(0, CHUNK, zrow, 0)
    for rr in range(640 // CHUNK):
        pltpu.sync_copy(b0, acc.at[pl.ds(sid * 640 + rr * CHUNK, CHUNK)])
    plsc.subcore_barrier()

    for j in range(5):
        e_start(j, j)
    e_wait(0); g_start(0, 0)
    g_wait(0); scale(0, 0); s_start(0, 0)
    e_start(5, 5)
    e_wait(1); g_start(1, 1)
    g_wait(1); scale(1, 1); s_start(1, 1); s_wait(0)
    e_start(6, 0)
    e_wait(2); g_start(2, 0)

    def main(i, _):
        j6 = 2 + i * 6
        for u in range(6):
            jj = j6 + u
            r = (2 + u) % 6
            b = u % 2
            g_wait(b)
            scale(r, b)
            s_start(r, b)
            s_wait(1 - b)

            @pl.when(jj + 5 < NC)
            def _():
                e_start(jj + 5, (r + 5) % 6)

            e_wait((r + 1) % 6)
            g_start((r + 1) % 6, 1 - b)
        return 0

    lax.fori_loop(0, (NC - 3) // 6, main, 0)

    g_wait(0); scale(2, 0); s_start(2, 0)
    s_wait(1); s_wait(0)
    plsc.subcore_barrier()
    pltpu.sync_copy(acc.at[pl.ds(sid * 640, 640)], out_hbm.at[cid, sid])


@functools.partial(
    pl.kernel,
    out_type=jax.ShapeDtypeStruct((2, NSUB, 640, 128), jnp.float32),
    mesh=_mesh,
    scratch_types=[
        pltpu.VMEM((6, CHUNK), jnp.int32),
        pltpu.VMEM((6, CHUNK), jnp.int32),
        pltpu.VMEM((6, CHUNK), jnp.float32),
        pltpu.VMEM((CHUNK, 128), jnp.float32),
        pltpu.VMEM((CHUNK, 128), jnp.float32),
        pltpu.VMEM_SHARED((NP, 128), jnp.float32),
        pltpu.SemaphoreType.DMA,
        pltpu.SemaphoreType.DMA,
        pltpu.SemaphoreType.DMA,
        pltpu.SemaphoreType.DMA,
        pltpu.SemaphoreType.DMA,
        pltpu.SemaphoreType.DMA,
        pltpu.SemaphoreType.DMA,
        pltpu.SemaphoreType.DMA,
        pltpu.SemaphoreType.DMA,
        pltpu.SemaphoreType.DMA,
        pltpu.SemaphoreType.DMA,
        pltpu.SemaphoreType.DMA,
    ],
)
def _agg_kernel(us_hbm, epr_hbm, epc_hbm, epw_hbm, out_hbm,
                err_, erc, erw, b0, b1, acc,
                es0, es1, es2, es3, es4, es5,
                gs0, gs1, gs2, ss0, ss1, ss2):
    cid = lax.axis_index("c")
    sid = lax.axis_index("s")
    bufs = (b0, b1)
    esems = (es0, es1, es2, es3, es4, es5)
    gsems = (gs0, gs1, gs2)
    ssems = (ss0, ss1, ss2)
    wid = cid * NSUB + sid
    zeros16 = jnp.zeros((16,), jnp.float32)

    def e_start(j, r):
        pltpu.async_copy(epr_hbm.at[wid, j], err_.at[r], esems[r])
        pltpu.async_copy(epc_hbm.at[wid, j], erc.at[r], esems[r])
        pltpu.async_copy(epw_hbm.at[wid, j], erw.at[r], esems[r])

    def e_wait(r):
        pltpu.make_async_copy(epr_hbm.at[wid, 0], err_.at[r], esems[r]).wait()
        pltpu.make_async_copy(epc_hbm.at[wid, 0], erc.at[r], esems[r]).wait()
        pltpu.make_async_copy(epw_hbm.at[wid, 0], erw.at[r], esems[r]).wait()

    def g_start(r, b):
        pltpu.async_copy(us_hbm.at[err_.at[r]], bufs[b], gsems[b])

    def g_wait(b):
        pltpu.make_async_copy(us_hbm.at[err_.at[0]], bufs[b], gsems[b]).wait()

    def s_start(r, b):
        pltpu.async_copy(bufs[b], acc.at[erc.at[r]], ssems[b], add=True)

    def s_wait(b):
        pltpu.make_async_copy(bufs[b], acc.at[erc.at[0]], ssems[b]).wait()

    def scale(r, b):
        buf = bufs[b]

        def escale(g, _):
            ew16 = erw[r, pl.ds(g * 16, 16)]
            for e16 in range(16):
                w = _bcast_lane(ew16, e16)
                e = g * 16 + e16
                for f in range(128 // 16):
                    sl = pl.ds(f * 16, 16)
                    buf[e, sl] = buf[e, sl] * w
            return 0

        lax.fori_loop(0, CHUNK // 16, escale, 0)

    def zrow(i, _):
        for f in range(128 // 16):
            b0[i, pl.ds(f * 16, 16)] = zeros16
        return 0

    lax.fori_loop(0, CHUNK, zrow, 0)
    for rr in range(640 // CHUNK):
        pltpu.sync_copy(b0, acc.at[pl.ds(sid * 640 + rr * CHUNK, CHUNK)])
    plsc.subcore_barrier()

    def simple(j, _):
        e_start(j, 0)
        e_wait(0)
        g_start(0, 0)
        g_wait(0)
        scale(0, 0)
        pltpu.sync_copy(b0, acc.at[erc.at[0]], add=True)
        return 0

    lax.fori_loop(0, NC, simple, 0)
    plsc.subcore_barrier()
    pltpu.sync_copy(acc.at[pl.ds(sid * 640, 640)], out_hbm.at[cid, sid])


def _aggregate(us, epr_t, epc_t, epw_t):
    o = _agg_kernel(us, epr_t, epc_t, epw_t)
    o = o.reshape(2, NP, 128)[:, :NN]
    return o[0] + o[1]


def _agg_full(u, dinv, epr_t, epc_t, epw_t):
    us = u * dinv[:, None]
    blocks = []
    for f in range(u.shape[1] // 128):
        usf = us[:, f * 128:(f + 1) * 128]
        blocks.append(_aggregate(usf, epr_t, epc_t, epw_t) + usf)
    acc = jnp.concatenate(blocks, axis=1) if len(blocks) > 1 else blocks[0]
    return acc * dinv[:, None]


def _batch_norm(z, g, be):
    mu = z.mean(axis=0)
    var = z.var(axis=0)
    return (z - mu) * lax.rsqrt(var + BN_EPS) * g + be


def kernel(x, edge_index, edge_attr, W1, b1, W2, b2, W3, b3, W4, b4, W5, b5,
           g1, be1, g2, be2, g3, be3, g4, be4):
    pad = EPAD - EE
    row = jnp.concatenate([edge_index[0], jnp.zeros((pad,), jnp.int32)])
    col = jnp.concatenate([edge_index[1], jnp.zeros((pad,), jnp.int32)])
    ew = jnp.concatenate([edge_attr, jnp.zeros((pad,), jnp.float32)])
    epr_t = row.reshape(2 * NSUB, NC, CHUNK)
    epc_t = col.reshape(2 * NSUB, NC, CHUNK)
    epw_t = ew.reshape(2 * NSUB, NC, CHUNK)
    col_d = col.reshape(2, NSUB, NCD, 128)
    ew_d = ew.reshape(2, NSUB, NCD, 128)

    degp = _deg_kernel(col_d, ew_d)
    deg = degp.reshape(2, NP)[:, :NN].sum(axis=0) + 1.0
    dinv = lax.rsqrt(deg)

    agg = lambda u: _agg_full(u, dinv, epr_t, epc_t, epw_t)

    h = jax.nn.sigmoid(_batch_norm(agg(x) @ W1 + b1, g1, be1))
    h = jax.nn.sigmoid(_batch_norm(agg(h) @ W2 + b2, g2, be2))
    h = jax.nn.sigmoid(_batch_norm(agg(h @ W3) + b3, g3, be3))
    h = jax.nn.sigmoid(_batch_norm(agg(h @ W4) + b4, g4, be4))
    h = jax.nn.sigmoid(agg(h @ W5) + b5)

    A = h.T @ h
    A = (A + A.T) / 2.0
    A = A - jnp.diag(jnp.diag(A))
    return A


# speedup vs baseline: 1.9279x; 1.9279x over previous
import functools

import jax
import jax.numpy as jnp
from jax import lax
from jax.experimental import pallas as pl
from jax.experimental.pallas import tpu as pltpu
from jax.experimental.pallas import tpu_sc as plsc

NN = 10000
NP = 10240
EE = 320000
NSUB = 16
CHUNK = 128
NCHUNK = 79
EPAD = 2 * NSUB * NCHUNK * CHUNK
BN_EPS = 1e-3

_mesh = plsc.VectorSubcoreMesh(core_axis_name="c", subcore_axis_name="s")

_BCAST_DNUMS = lax.GatherDimensionNumbers(
    offset_dims=(), collapsed_slice_dims=(0,), start_index_map=(0,))


def _bcast_lane(v16, lane):
    idx = jnp.full((16, 1), lane, jnp.int32)
    return lax.gather(v16, idx, _BCAST_DNUMS, (1,),
                      mode=lax.GatherScatterMode.PROMISE_IN_BOUNDS)


@functools.partial(
    pl.kernel,
    out_type=jax.ShapeDtypeStruct((2, NSUB, 640), jnp.float32),
    mesh=_mesh,
    scratch_types=[
        pltpu.VMEM((NCHUNK, CHUNK), jnp.int32),
        pltpu.VMEM((NCHUNK, CHUNK), jnp.float32),
        pltpu.VMEM((640,), jnp.float32),
        pltpu.VMEM_SHARED((NP,), jnp.float32),
    ],
)
def _deg_kernel(col_hbm, ew_hbm, out_hbm, col_v, ew_v, zb, acc):
    cid = lax.axis_index("c")
    sid = lax.axis_index("s")
    wid = cid * NSUB + sid
    zeros16 = jnp.zeros((16,), jnp.float32)

    def zb_body(i, _):
        zb[pl.ds(i * 16, 16)] = zeros16
        return 0

    lax.fori_loop(0, 640 // 16, zb_body, 0)
    pltpu.sync_copy(zb, acc.at[pl.ds(sid * 640, 640)])
    pltpu.sync_copy(col_hbm.at[wid], col_v)
    pltpu.sync_copy(ew_hbm.at[wid], ew_v)
    plsc.subcore_barrier()

    def chunk_body(j, _):
        pltpu.sync_copy(ew_v.at[j], acc.at[col_v.at[j]], add=True)
        return 0

    lax.fori_loop(0, NCHUNK, chunk_body, 0)
    plsc.subcore_barrier()
    pltpu.sync_copy(acc.at[pl.ds(sid * 640, 640)], out_hbm.at[cid, sid])


@functools.partial(
    pl.kernel,
    out_type=jax.ShapeDtypeStruct((2, NSUB, 640, CHUNK), jnp.float32),
    mesh=_mesh,
    scratch_types=[
        pltpu.VMEM((NCHUNK, CHUNK), jnp.int32),
        pltpu.VMEM((NCHUNK, CHUNK), jnp.int32),
        pltpu.VMEM((NCHUNK, CHUNK), jnp.float32),
        pltpu.VMEM((CHUNK, CHUNK), jnp.float32),
        pltpu.VMEM_SHARED((NP, CHUNK), jnp.float32),
        pltpu.SemaphoreType.DMA,
    ],
)
def _agg_kernel(us_hbm, row_hbm, col_hbm, ew_hbm, out_hbm,
                row_v, col_v, ew_v, rows_v, acc, sem):
    cid = lax.axis_index("c")
    sid = lax.axis_index("s")
    wid = cid * NSUB + sid
    zeros16 = jnp.zeros((16,), jnp.float32)

    def zrow(i, _):
        for f in range(CHUNK // 16):
            rows_v[i, pl.ds(f * 16, 16)] = zeros16
        return 0

    lax.fori_loop(0, CHUNK, zrow, 0)
    for r in range(640 // CHUNK):
        pltpu.sync_copy(rows_v, acc.at[pl.ds(sid * 640 + r * CHUNK, CHUNK)])
    pltpu.sync_copy(row_hbm.at[wid], row_v)
    pltpu.sync_copy(col_hbm.at[wid], col_v)
    pltpu.sync_copy(ew_hbm.at[wid], ew_v)
    plsc.subcore_barrier()

    def chunk_body(j, _):
        pltpu.async_copy(us_hbm.at[row_v.at[j]], rows_v, sem).wait()

        def escale(g, _):
            ew16 = ew_v[j, pl.ds(g * 16, 16)]
            for e16 in range(16):
                w = _bcast_lane(ew16, e16)
                e = g * 16 + e16
                for f in range(CHUNK // 16):
                    sl = pl.ds(f * 16, 16)
                    rows_v[e, sl] = rows_v[e, sl] * w
            return 0

        lax.fori_loop(0, CHUNK // 16, escale, 0)
        pltpu.sync_copy(rows_v, acc.at[col_v.at[j]], add=True)
        return 0

    lax.fori_loop(0, NCHUNK, chunk_body, 0)
    plsc.subcore_barrier()
    pltpu.sync_copy(acc.at[pl.ds(sid * 640, 640)], out_hbm.at[cid, sid])


def _aggregate(us, row_t, col_t, ew_t):
    o = _agg_kernel(us, row_t, col_t, ew_t)
    o = o.reshape(2, NP, CHUNK)[:, :NN]
    return o[0] + o[1]


def _agg_full(u, dinv, row_t, col_t, ew_t):
    us = u * dinv[:, None]
    blocks = []
    for f in range(u.shape[1] // CHUNK):
        usf = us[:, f * CHUNK:(f + 1) * CHUNK]
        blocks.append(_aggregate(usf, row_t, col_t, ew_t) + usf)
    acc = jnp.concatenate(blocks, axis=1) if len(blocks) > 1 else blocks[0]
    return acc * dinv[:, None]


def _batch_norm(z, g, be):
    mu = z.mean(axis=0)
    var = z.var(axis=0)
    return (z - mu) * lax.rsqrt(var + BN_EPS) * g + be


def kernel(x, edge_index, edge_attr, W1, b1, W2, b2, W3, b3, W4, b4, W5, b5,
           g1, be1, g2, be2, g3, be3, g4, be4):
    pad = EPAD - EE
    row = jnp.concatenate([edge_index[0], jnp.zeros((pad,), jnp.int32)])
    col = jnp.concatenate([edge_index[1], jnp.zeros((pad,), jnp.int32)])
    ew = jnp.concatenate([edge_attr, jnp.zeros((pad,), jnp.float32)])
    row_t = row.reshape(2 * NSUB, NCHUNK, CHUNK)
    col_t = col.reshape(2 * NSUB, NCHUNK, CHUNK)
    ew_t = ew.reshape(2 * NSUB, NCHUNK, CHUNK)

    degp = _deg_kernel(col_t, ew_t)
    deg = degp.reshape(2, NP)[:, :NN].sum(axis=0) + 1.0
    dinv = lax.rsqrt(deg)

    agg = lambda u: _agg_full(u, dinv, row_t, col_t, ew_t)

    h = jax.nn.sigmoid(_batch_norm(agg(x) @ W1 + b1, g1, be1))
    h = jax.nn.sigmoid(_batch_norm(agg(h) @ W2 + b2, g2, be2))
    h = jax.nn.sigmoid(_batch_norm(agg(h @ W3) + b3, g3, be3))
    h = jax.nn.sigmoid(_batch_norm(agg(h @ W4) + b4, g4, be4))
    h = jax.nn.sigmoid(agg(h @ W5) + b5)

    A = h.T @ h
    A = (A + A.T) / 2.0
    A = A - jnp.diag(jnp.diag(A))
    return A
